# baseline (speedup 1.0000x reference)
import functools

import jax
import jax.numpy as jnp
from jax import lax
from jax.experimental import pallas as pl
from jax.experimental.pallas import tpu as pltpu
from jax.experimental.pallas import tpu_sc as plsc

N, E, D, DO, DE = 10000, 160000, 256, 256, 16
K = 16
EPT = E // 16
NCHUNK = EPT // K
NPAD = 10240
ROWS_PT = NPAD // 16


def _pack64(half):
    lo = lax.bitcast_convert_type(half[:, 0:64].astype(jnp.bfloat16),
                                  jnp.uint16).astype(jnp.uint32)
    hi = lax.bitcast_convert_type(half[:, 64:128].astype(jnp.bfloat16),
                                  jnp.uint16).astype(jnp.uint32)
    return lax.bitcast_convert_type(lo | (hi << 16), jnp.int32)


def _pack_bf16_pairs(half):
    packed = _pack64(half)
    return jnp.concatenate(
        [packed, jnp.zeros(packed.shape, jnp.int32)], axis=1)


def _tc1_body(x_ref, wn1_ref, bn1_ref, wn2_ref, bn2_ref, we1_ref,
              h_ref, a0_ref, a1_ref, b0_ref, b1_ref):
    xb = x_ref[...]
    t = jnp.maximum(jnp.dot(xb, wn1_ref[...],
                            preferred_element_type=jnp.float32)
                    + bn1_ref[...], 0.0)
    h_ref[...] = jnp.dot(t.astype(jnp.bfloat16), wn2_ref[...],
                         preferred_element_type=jnp.float32) + bn2_ref[...]
    a = jnp.dot(xb, we1_ref[0:D, :], preferred_element_type=jnp.float32)
    a0_ref[...] = _pack_bf16_pairs(a[:, 0:128])
    a1_ref[...] = _pack_bf16_pairs(a[:, 128:256])
    b = jnp.dot(xb, we1_ref[D:2 * D, :], preferred_element_type=jnp.float32)
    b0_ref[...] = _pack_bf16_pairs(b[:, 0:128])
    b1_ref[...] = _pack_bf16_pairs(b[:, 128:256])


def _tc1(x, Wn1, bn1, Wn2, bn2, We1):
    BN = 2000
    f32 = jnp.float32
    i32 = jnp.int32
    return pl.pallas_call(
        _tc1_body,
        grid=(N // BN,),
        in_specs=[
            pl.BlockSpec((BN, D), lambda i: (i, 0)),
            pl.BlockSpec((D, DO), lambda i: (0, 0)),
            pl.BlockSpec((1, DO), lambda i: (0, 0)),
            pl.BlockSpec((DO, DO), lambda i: (0, 0)),
            pl.BlockSpec((1, DO), lambda i: (0, 0)),
            pl.BlockSpec((2 * D + DE, DO), lambda i: (0, 0)),
        ],
        out_specs=[
            pl.BlockSpec((BN, DO), lambda i: (i, 0)),
            pl.BlockSpec((BN, 128), lambda i: (i, 0)),
            pl.BlockSpec((BN, 128), lambda i: (i, 0)),
            pl.BlockSpec((BN, 128), lambda i: (i, 0)),
            pl.BlockSpec((BN, 128), lambda i: (i, 0)),
        ],
        out_shape=[
            jax.ShapeDtypeStruct((N, DO), f32),
            jax.ShapeDtypeStruct((N, 128), i32),
            jax.ShapeDtypeStruct((N, 128), i32),
            jax.ShapeDtypeStruct((N, 128), i32),
            jax.ShapeDtypeStruct((N, 128), i32),
        ],
    )(x.astype(jnp.bfloat16), Wn1.astype(jnp.bfloat16), bn1.reshape(1, DO),
      Wn2.astype(jnp.bfloat16), bn2.reshape(1, DO),
      We1.astype(jnp.bfloat16))


def _tc2_body(ea2_ref, we1c_ref, be1_ref, c_ref):
    ea2 = ea2_ref[...]
    cl = jnp.dot(ea2[:, 0:DE], we1c_ref[...],
                 preferred_element_type=jnp.float32) + be1_ref[...]
    cr = jnp.dot(ea2[:, DE:2 * DE], we1c_ref[...],
                 preferred_element_type=jnp.float32) + be1_ref[...]
    c_ref[...] = jnp.concatenate([_pack64(cl), _pack64(cr)], axis=1)


def _tc2(ea2, We1, be1):
    BR = 5000
    EB = (E // 2) // BR
    return pl.pallas_call(
        _tc2_body,
        grid=(2, EB),
        in_specs=[
            pl.BlockSpec((BR, 2 * DE), lambda h, i: (i, 0)),
            pl.BlockSpec((DE, 128), lambda h, i: (2 * D // DE, h)),
            pl.BlockSpec((1, 128), lambda h, i: (0, h)),
        ],
        out_specs=pl.BlockSpec((BR, 128), lambda h, i: (h * EB + i, 0)),
        out_shape=jax.ShapeDtypeStruct((E, 128), jnp.int32),
    )(ea2, We1, be1.reshape(1, DO))


def _sc_body(a0_hbm, a1_hbm, b0_hbm, b1_hbm, c_hbm, src_hbm, dst_hbm,
             s_out0, s_out1, deg_out,
             sring, dring,
             rows_a0, rows_a1, rows_b0, rows_b1, rows_c0, rows_c1,
             upd0, upd1, ones_v, zvec, s_sh, deg_sh,
             isemE, isemO, gsem0, gsem1, ssem0, ssem1, dsem):
    c = lax.axis_index("c")
    s = lax.axis_index("s")
    rows_a = (rows_a0, rows_a1)
    rows_b = (rows_b0, rows_b1)
    rows_c = (rows_c0, rows_c1)
    upd = (upd0, upd1)
    gsem = (gsem0, gsem1)
    ssem = (ssem0, ssem1)
    zero16 = jnp.zeros((16,), jnp.float32)
    one16 = jnp.ones((16,), jnp.float32)

    ones_v[pl.ds(0, 16)] = one16

    def _zv(r, carry):
        zvec[pl.ds(r * 16, 16)] = zero16
        return carry
    lax.fori_loop(0, ROWS_PT // 16, _zv, 0)

    def _zrow(r, carry):
        for k in range(128 // 16):
            upd0[r, pl.ds(k * 16, 16)] = zero16
            upd1[r, pl.ds(k * 16, 16)] = zero16
        return carry
    lax.fori_loop(0, K, _zrow, 0)

    for j in range(ROWS_PT // K):
        pltpu.sync_copy(upd0, s_sh.at[pl.ds(s * ROWS_PT + j * K, K)])

    @pl.when(c == 0)
    def _():
        pltpu.sync_copy(zvec, deg_sh.at[pl.ds(s * ROWS_PT, ROWS_PT)])

    plsc.subcore_barrier()

    ebase = s * EPT

    def fetch_idx(t, sem):
        js = lax.rem(t, 3)
        jd = lax.rem(t, 6)
        pltpu.async_copy(src_hbm.at[pl.ds(ebase + t * K, K)],
                         sring.at[js], sem)
        pltpu.async_copy(dst_hbm.at[pl.ds(ebase + t * K, K)],
                         dring.at[jd], sem)

    def wait_idx(t, sem):
        js = lax.rem(t, 3)
        jd = lax.rem(t, 6)
        pltpu.make_async_copy(src_hbm.at[pl.ds(ebase + t * K, K)],
                              sring.at[js], sem).wait()
        pltpu.make_async_copy(dst_hbm.at[pl.ds(ebase + t * K, K)],
                              dring.at[jd], sem).wait()

    def issue_gathers(t, b):
        js = lax.rem(t, 3)
        jd = lax.rem(t, 6)

        @pl.when(c == 0)
        def _():
            pltpu.async_copy(a0_hbm.at[sring.at[js]], rows_a[b], gsem[b])
            pltpu.async_copy(b0_hbm.at[dring.at[jd]], rows_b[b], gsem[b])

        @pl.when(c == 1)
        def _():
            pltpu.async_copy(a1_hbm.at[sring.at[js]], rows_a[b], gsem[b])
            pltpu.async_copy(b1_hbm.at[dring.at[jd]], rows_b[b], gsem[b])
        pltpu.async_copy(
            c_hbm.at[pl.ds(c * (E // 2) + s * (EPT // 2) + t * (K // 2),
                           K // 2)],
            rows_c[b], gsem[b])

    def wait_gathers(t, b):
        js = lax.rem(t, 3)
        jd = lax.rem(t, 6)
        pltpu.make_async_copy(a0_hbm.at[sring.at[js]], rows_a[b],
                              gsem[b]).wait()
        pltpu.make_async_copy(b0_hbm.at[dring.at[jd]], rows_b[b],
                              gsem[b]).wait()
        pltpu.make_async_copy(
            c_hbm.at[pl.ds(c * (E // 2) + s * (EPT // 2) + t * (K // 2),
                           K // 2)],
            rows_c[b], gsem[b]).wait()

    himask = jnp.full((16,), -65536, jnp.int32)

    def do_chunk(t, b):
        wait_gathers(t, b)
        jd = lax.rem(t, 6)
        pltpu.make_async_copy(upd[b], s_sh.at[dring.at[jd]], ssem[b]).wait()
        ra, rb, rc, u = rows_a[b], rows_b[b], rows_c[b], upd[b]

        def _row(q, cc):
            for p in range(2):
                r = 2 * q + p
                for m in range(4):
                    sl = pl.ds(m * 16, 16)
                    wa = ra[r, sl]
                    wb = rb[r, sl]
                    wc = rc[q, pl.ds(64 * p + m * 16, 16)]
                    la = lax.bitcast_convert_type(wa << 16, jnp.float32)
                    lb = lax.bitcast_convert_type(wb << 16, jnp.float32)
                    lc = lax.bitcast_convert_type(wc << 16, jnp.float32)
                    ha = lax.bitcast_convert_type(wa & himask, jnp.float32)
                    hb = lax.bitcast_convert_type(wb & himask, jnp.float32)
                    hc = lax.bitcast_convert_type(wc & himask, jnp.float32)
                    u[r, pl.ds(m * 16, 16)] = jnp.maximum(la + lb + lc, 0.0)
                    u[r, pl.ds(64 + m * 16, 16)] = jnp.maximum(
                        ha + hb + hc, 0.0)
            return cc
        lax.fori_loop(0, K // 2, _row, 0)
        pltpu.async_copy(u, s_sh.at[dring.at[jd]], ssem[b], add=True)

        @pl.when(c == 0)
        def _():
            pltpu.make_async_copy(ones_v, deg_sh.at[dring.at[jd]],
                                  dsem).wait()
            pltpu.async_copy(ones_v, deg_sh.at[dring.at[jd]], dsem, add=True)

    fetch_idx(0, isemE)
    fetch_idx(1, isemO)
    wait_idx(0, isemE)
    wait_idx(1, isemO)
    issue_gathers(0, 0)
    issue_gathers(1, 1)
    fetch_idx(2, isemE)
    pltpu.async_copy(upd0, s_sh.at[dring.at[0]], ssem0, add=True)
    pltpu.async_copy(upd1, s_sh.at[dring.at[1]], ssem1, add=True)

    @pl.when(c == 0)
    def _():
        pltpu.async_copy(zvec.at[pl.ds(0, K)], deg_sh.at[dring.at[0]],
                         dsem, add=True)

    NP2 = NCHUNK // 2

    def _pair(g, carry):
        t0 = 2 * g
        t1 = t0 + 1

        wait_idx(t0 + 2, isemE)
        do_chunk(t0, 0)
        issue_gathers(t0 + 2, 0)

        @pl.when(g < NP2 - 1)
        def _():
            fetch_idx(t0 + 3, isemO)
            wait_idx(t1 + 2, isemO)
        do_chunk(t1, 1)

        @pl.when(g < NP2 - 1)
        def _():
            issue_gathers(t1 + 2, 1)
            fetch_idx(t1 + 3, isemE)
        return carry

    lax.fori_loop(0, NP2, _pair, 0)
    do_chunk(NCHUNK - 1, 0)

    pltpu.make_async_copy(upd0, s_sh.at[dring.at[0]], ssem0).wait()
    pltpu.make_async_copy(upd1, s_sh.at[dring.at[0]], ssem1).wait()

    @pl.when(c == 0)
    def _():
        pltpu.make_async_copy(ones_v, deg_sh.at[dring.at[0]], dsem).wait()

    plsc.subcore_barrier()

    @pl.when(c == 0)
    def _():
        pltpu.sync_copy(s_sh.at[pl.ds(s * ROWS_PT, ROWS_PT)],
                        s_out0.at[pl.ds(s * ROWS_PT, ROWS_PT)])
        pltpu.sync_copy(deg_sh.at[pl.ds(s * ROWS_PT, ROWS_PT)],
                        deg_out.at[pl.ds(s * ROWS_PT, ROWS_PT)])

    @pl.when(c == 1)
    def _():
        pltpu.sync_copy(s_sh.at[pl.ds(s * ROWS_PT, ROWS_PT)],
                        s_out1.at[pl.ds(s * ROWS_PT, ROWS_PT)])


def _sc_edge(a0, a1, b0, b1, c_cat, src, dst):
    f32 = jnp.float32
    i32 = jnp.int32
    fn = functools.partial(
        pl.kernel,
        mesh=plsc.VectorSubcoreMesh(core_axis_name="c", subcore_axis_name="s"),
        out_type=[
            jax.ShapeDtypeStruct((NPAD, 128), f32),
            jax.ShapeDtypeStruct((NPAD, 128), f32),
            jax.ShapeDtypeStruct((NPAD,), f32),
        ],
        scratch_types=[
            pltpu.VMEM((3, K), i32),
            pltpu.VMEM((6, K), i32),
            pltpu.VMEM((K, 128), i32),
            pltpu.VMEM((K, 128), i32),
            pltpu.VMEM((K, 128), i32),
            pltpu.VMEM((K, 128), i32),
            pltpu.VMEM((K // 2, 128), i32),
            pltpu.VMEM((K // 2, 128), i32),
            pltpu.VMEM((K, 128), f32),
            pltpu.VMEM((K, 128), f32),
            pltpu.VMEM((K,), f32),
            pltpu.VMEM((ROWS_PT,), f32),
            pltpu.VMEM_SHARED((NPAD, 128), f32),
            pltpu.VMEM_SHARED((NPAD,), f32),
            pltpu.SemaphoreType.DMA,
            pltpu.SemaphoreType.DMA,
            pltpu.SemaphoreType.DMA,
            pltpu.SemaphoreType.DMA,
            pltpu.SemaphoreType.DMA,
            pltpu.SemaphoreType.DMA,
            pltpu.SemaphoreType.DMA,
        ],
    )(_sc_body)
    return fn(a0, a1, b0, b1, c_cat, src, dst)


def _tc3_body(h_ref, s0_ref, s1_ref, we2_ref, wu_ref, bu_ref, out_ref):
    bf = jnp.bfloat16
    agg = (jnp.dot(s0_ref[...].astype(bf), we2_ref[0:128, :],
                   preferred_element_type=jnp.float32)
           + jnp.dot(s1_ref[...].astype(bf), we2_ref[128:256, :],
                     preferred_element_type=jnp.float32))
    out_ref[...] = (jnp.dot(h_ref[...].astype(bf), wu_ref[0:DO, :],
                            preferred_element_type=jnp.float32)
                    + jnp.dot(agg.astype(bf), wu_ref[DO:2 * DO, :],
                              preferred_element_type=jnp.float32)
                    + bu_ref[...])


def _tc3(h_nodes, s0, s1, We2, Wu, bu):
    BN = 1000
    grid = (N // BN,)
    return pl.pallas_call(
        _tc3_body,
        grid=grid,
        in_specs=[
            pl.BlockSpec((BN, DO), lambda i: (i, 0)),
            pl.BlockSpec((BN, 128), lambda i: (i, 0)),
            pl.BlockSpec((BN, 128), lambda i: (i, 0)),
            pl.BlockSpec((DO, DO), lambda i: (0, 0)),
            pl.BlockSpec((2 * DO, DO), lambda i: (0, 0)),
            pl.BlockSpec((1, DO), lambda i: (0, 0)),
        ],
        out_specs=pl.BlockSpec((BN, DO), lambda i: (i, 0)),
        out_shape=jax.ShapeDtypeStruct((N, DO), jnp.float32),
    )(h_nodes, s0, s1, We2.astype(jnp.bfloat16), Wu.astype(jnp.bfloat16),
      bu.reshape(1, DO))


def kernel(x, edge_index, edge_attr, pos, Wn1, bn1, Wn2, bn2,
           We1, be1, We2, be2, Wu, bu):
    h_nodes, a0, a1, b0, b1 = _tc1(x, Wn1, bn1, Wn2, bn2, We1)
    src = edge_index[0]
    dst = edge_index[1]
    ea2 = edge_attr.reshape(E // 2, 2 * DE)
    c_cat = _tc2(ea2, We1, be1)
    s_acc0, s_acc1, deg = _sc_edge(a0, a1, b0, b1, c_cat, src, dst)
    h_out = _tc3(h_nodes, s_acc0, s_acc1, We2, Wu, bu)
    h_out = h_out + deg[:N, None] * jnp.dot(be2, Wu[DO:2 * DO, :])
    return h_out


# speedup vs baseline: 1.1166x; 1.1166x over previous
import functools

import jax
import jax.numpy as jnp
from jax import lax
from jax.experimental import pallas as pl
from jax.experimental.pallas import tpu as pltpu
from jax.experimental.pallas import tpu_sc as plsc

N, E, D, DO, DE = 10000, 160000, 256, 256, 16
K = 16
EPT = E // 16
NCHUNK = EPT // K
NPAD = 10240
ROWS_PT = NPAD // 16


def _pack64(half):
    lo = lax.bitcast_convert_type(half[:, 0:64].astype(jnp.bfloat16),
                                  jnp.uint16).astype(jnp.uint32)
    hi = lax.bitcast_convert_type(half[:, 64:128].astype(jnp.bfloat16),
                                  jnp.uint16).astype(jnp.uint32)
    return lax.bitcast_convert_type(lo | (hi << 16), jnp.int32)


def _pack_bf16_pairs(half):
    packed = _pack64(half)
    return jnp.concatenate(
        [packed, jnp.zeros(packed.shape, jnp.int32)], axis=1)


def _tc1_body(x_ref, wn1_ref, bn1_ref, wn2_ref, bn2_ref, we1_ref,
              h_ref, a0_ref, a1_ref, b0_ref, b1_ref):
    xb = x_ref[...]
    t = jnp.maximum(jnp.dot(xb, wn1_ref[...],
                            preferred_element_type=jnp.float32)
                    + bn1_ref[...], 0.0)
    h_ref[...] = jnp.dot(t, wn2_ref[...],
                         preferred_element_type=jnp.float32) + bn2_ref[...]
    a = jnp.dot(xb, we1_ref[0:D, :], preferred_element_type=jnp.float32)
    a0_ref[...] = _pack_bf16_pairs(a[:, 0:128])
    a1_ref[...] = _pack_bf16_pairs(a[:, 128:256])
    b = jnp.dot(xb, we1_ref[D:2 * D, :], preferred_element_type=jnp.float32)
    b0_ref[...] = _pack_bf16_pairs(b[:, 0:128])
    b1_ref[...] = _pack_bf16_pairs(b[:, 128:256])


def _tc1(x, Wn1, bn1, Wn2, bn2, We1):
    BN = 1000
    f32 = jnp.float32
    i32 = jnp.int32
    return pl.pallas_call(
        _tc1_body,
        grid=(N // BN,),
        in_specs=[
            pl.BlockSpec((BN, D), lambda i: (i, 0)),
            pl.BlockSpec((D, DO), lambda i: (0, 0)),
            pl.BlockSpec((1, DO), lambda i: (0, 0)),
            pl.BlockSpec((DO, DO), lambda i: (0, 0)),
            pl.BlockSpec((1, DO), lambda i: (0, 0)),
            pl.BlockSpec((2 * D + DE, DO), lambda i: (0, 0)),
        ],
        out_specs=[
            pl.BlockSpec((BN, DO), lambda i: (i, 0)),
            pl.BlockSpec((BN, 128), lambda i: (i, 0)),
            pl.BlockSpec((BN, 128), lambda i: (i, 0)),
            pl.BlockSpec((BN, 128), lambda i: (i, 0)),
            pl.BlockSpec((BN, 128), lambda i: (i, 0)),
        ],
        out_shape=[
            jax.ShapeDtypeStruct((N, DO), f32),
            jax.ShapeDtypeStruct((N, 128), i32),
            jax.ShapeDtypeStruct((N, 128), i32),
            jax.ShapeDtypeStruct((N, 128), i32),
            jax.ShapeDtypeStruct((N, 128), i32),
        ],
    )(x, Wn1, bn1.reshape(1, DO), Wn2, bn2.reshape(1, DO), We1)


def _tc2_body(ea_ref, we1c_ref, be1_ref, c_ref):
    ce = jnp.dot(ea_ref[...], we1c_ref[...],
                 preferred_element_type=jnp.float32) + be1_ref[...]
    c_ref[...] = _pack_bf16_pairs(ce)


def _tc2(edge_attr, We1, be1):
    BE = 2000
    EB = E // BE
    return pl.pallas_call(
        _tc2_body,
        grid=(2, EB),
        in_specs=[
            pl.BlockSpec((BE, DE), lambda h, i: (i, 0)),
            pl.BlockSpec((DE, 128), lambda h, i: (2 * D // DE, h)),
            pl.BlockSpec((1, 128), lambda h, i: (0, h)),
        ],
        out_specs=pl.BlockSpec((BE, 128), lambda h, i: (h * EB + i, 0)),
        out_shape=jax.ShapeDtypeStruct((2 * E, 128), jnp.int32),
    )(edge_attr, We1, be1.reshape(1, DO))


def _sc_body(a0_hbm, a1_hbm, b0_hbm, b1_hbm, c_hbm, src_hbm, dst_hbm,
             s_out0, s_out1, deg_out,
             sring, dring,
             rows_a0, rows_a1, rows_b0, rows_b1, rows_c0, rows_c1,
             upd0, upd1, ones_v, zvec, s_sh, deg_sh,
             isemE, isemO, gsem0, gsem1, ssem0, ssem1, dsem):
    c = lax.axis_index("c")
    s = lax.axis_index("s")
    rows_a = (rows_a0, rows_a1)
    rows_b = (rows_b0, rows_b1)
    rows_c = (rows_c0, rows_c1)
    upd = (upd0, upd1)
    gsem = (gsem0, gsem1)
    ssem = (ssem0, ssem1)
    zero16 = jnp.zeros((16,), jnp.float32)
    one16 = jnp.ones((16,), jnp.float32)

    ones_v[pl.ds(0, 16)] = one16

    def _zv(r, carry):
        zvec[pl.ds(r * 16, 16)] = zero16
        return carry
    lax.fori_loop(0, ROWS_PT // 16, _zv, 0)

    def _zrow(r, carry):
        for k in range(128 // 16):
            upd0[r, pl.ds(k * 16, 16)] = zero16
            upd1[r, pl.ds(k * 16, 16)] = zero16
        return carry
    lax.fori_loop(0, K, _zrow, 0)

    for j in range(ROWS_PT // K):
        pltpu.sync_copy(upd0, s_sh.at[pl.ds(s * ROWS_PT + j * K, K)])

    @pl.when(c == 0)
    def _():
        pltpu.sync_copy(zvec, deg_sh.at[pl.ds(s * ROWS_PT, ROWS_PT)])

    plsc.subcore_barrier()

    ebase = s * EPT

    def fetch_idx(t, sem):
        js = lax.rem(t, 3)
        jd = lax.rem(t, 6)
        pltpu.async_copy(src_hbm.at[pl.ds(ebase + t * K, K)],
                         sring.at[js], sem)
        pltpu.async_copy(dst_hbm.at[pl.ds(ebase + t * K, K)],
                         dring.at[jd], sem)

    def wait_idx(t, sem):
        js = lax.rem(t, 3)
        jd = lax.rem(t, 6)
        pltpu.make_async_copy(src_hbm.at[pl.ds(ebase + t * K, K)],
                              sring.at[js], sem).wait()
        pltpu.make_async_copy(dst_hbm.at[pl.ds(ebase + t * K, K)],
                              dring.at[jd], sem).wait()

    def issue_gathers(t, b):
        js = lax.rem(t, 3)
        jd = lax.rem(t, 6)

        @pl.when(c == 0)
        def _():
            pltpu.async_copy(a0_hbm.at[sring.at[js]], rows_a[b], gsem[b])
            pltpu.async_copy(b0_hbm.at[dring.at[jd]], rows_b[b], gsem[b])

        @pl.when(c == 1)
        def _():
            pltpu.async_copy(a1_hbm.at[sring.at[js]], rows_a[b], gsem[b])
            pltpu.async_copy(b1_hbm.at[dring.at[jd]], rows_b[b], gsem[b])
        pltpu.async_copy(c_hbm.at[pl.ds(c * E + ebase + t * K, K)],
                         rows_c[b], gsem[b])

    def wait_gathers(t, b):
        js = lax.rem(t, 3)
        jd = lax.rem(t, 6)
        pltpu.make_async_copy(a0_hbm.at[sring.at[js]], rows_a[b],
                              gsem[b]).wait()
        pltpu.make_async_copy(b0_hbm.at[dring.at[jd]], rows_b[b],
                              gsem[b]).wait()
        pltpu.make_async_copy(c_hbm.at[pl.ds(c * E + ebase + t * K, K)],
                              rows_c[b], gsem[b]).wait()

    himask = jnp.full((16,), -65536, jnp.int32)

    def do_chunk(t, b):
        wait_gathers(t, b)
        jd = lax.rem(t, 6)
        pltpu.make_async_copy(upd[b], s_sh.at[dring.at[jd]], ssem[b]).wait()
        ra, rb, rc, u = rows_a[b], rows_b[b], rows_c[b], upd[b]

        def _row(q, cc):
            for p in range(2):
                r = 2 * q + p
                for m in range(4):
                    sl = pl.ds(m * 16, 16)
                    wa = ra[r, sl]
                    wb = rb[r, sl]
                    wc = rc[r, sl]
                    la = lax.bitcast_convert_type(wa << 16, jnp.float32)
                    lb = lax.bitcast_convert_type(wb << 16, jnp.float32)
                    lc = lax.bitcast_convert_type(wc << 16, jnp.float32)
                    ha = lax.bitcast_convert_type(wa & himask, jnp.float32)
                    hb = lax.bitcast_convert_type(wb & himask, jnp.float32)
                    hc = lax.bitcast_convert_type(wc & himask, jnp.float32)
                    u[r, pl.ds(m * 16, 16)] = jnp.maximum(la + lb + lc, 0.0)
                    u[r, pl.ds(64 + m * 16, 16)] = jnp.maximum(
                        ha + hb + hc, 0.0)
            return cc
        lax.fori_loop(0, K // 2, _row, 0)
        pltpu.async_copy(u, s_sh.at[dring.at[jd]], ssem[b], add=True)

        @pl.when(c == 0)
        def _():
            pltpu.make_async_copy(ones_v, deg_sh.at[dring.at[jd]],
                                  dsem).wait()
            pltpu.async_copy(ones_v, deg_sh.at[dring.at[jd]], dsem, add=True)

    fetch_idx(0, isemE)
    fetch_idx(1, isemO)
    wait_idx(0, isemE)
    wait_idx(1, isemO)
    issue_gathers(0, 0)
    issue_gathers(1, 1)
    fetch_idx(2, isemE)
    pltpu.async_copy(upd0, s_sh.at[dring.at[0]], ssem0, add=True)
    pltpu.async_copy(upd1, s_sh.at[dring.at[1]], ssem1, add=True)

    @pl.when(c == 0)
    def _():
        pltpu.async_copy(zvec.at[pl.ds(0, K)], deg_sh.at[dring.at[0]],
                         dsem, add=True)

    NP2 = NCHUNK // 2

    def _pair(g, carry):
        t0 = 2 * g
        t1 = t0 + 1

        wait_idx(t0 + 2, isemE)
        do_chunk(t0, 0)
        issue_gathers(t0 + 2, 0)

        @pl.when(g < NP2 - 1)
        def _():
            fetch_idx(t0 + 3, isemO)
            wait_idx(t1 + 2, isemO)
        do_chunk(t1, 1)

        @pl.when(g < NP2 - 1)
        def _():
            issue_gathers(t1 + 2, 1)
            fetch_idx(t1 + 3, isemE)
        return carry

    lax.fori_loop(0, NP2, _pair, 0)
    do_chunk(NCHUNK - 1, 0)

    pltpu.make_async_copy(upd0, s_sh.at[dring.at[0]], ssem0).wait()
    pltpu.make_async_copy(upd1, s_sh.at[dring.at[0]], ssem1).wait()

    @pl.when(c == 0)
    def _():
        pltpu.make_async_copy(ones_v, deg_sh.at[dring.at[0]], dsem).wait()

    plsc.subcore_barrier()

    @pl.when(c == 0)
    def _():
        pltpu.sync_copy(s_sh.at[pl.ds(s * ROWS_PT, ROWS_PT)],
                        s_out0.at[pl.ds(s * ROWS_PT, ROWS_PT)])
        pltpu.sync_copy(deg_sh.at[pl.ds(s * ROWS_PT, ROWS_PT)],
                        deg_out.at[pl.ds(s * ROWS_PT, ROWS_PT)])

    @pl.when(c == 1)
    def _():
        pltpu.sync_copy(s_sh.at[pl.ds(s * ROWS_PT, ROWS_PT)],
                        s_out1.at[pl.ds(s * ROWS_PT, ROWS_PT)])


def _sc_edge(a0, a1, b0, b1, c_cat, src, dst):
    f32 = jnp.float32
    i32 = jnp.int32
    fn = functools.partial(
        pl.kernel,
        mesh=plsc.VectorSubcoreMesh(core_axis_name="c", subcore_axis_name="s"),
        out_type=[
            jax.ShapeDtypeStruct((NPAD, 128), f32),
            jax.ShapeDtypeStruct((NPAD, 128), f32),
            jax.ShapeDtypeStruct((NPAD,), f32),
        ],
        scratch_types=[
            pltpu.VMEM((3, K), i32),
            pltpu.VMEM((6, K), i32),
            pltpu.VMEM((K, 128), i32),
            pltpu.VMEM((K, 128), i32),
            pltpu.VMEM((K, 128), i32),
            pltpu.VMEM((K, 128), i32),
            pltpu.VMEM((K, 128), i32),
            pltpu.VMEM((K, 128), i32),
            pltpu.VMEM((K, 128), f32),
            pltpu.VMEM((K, 128), f32),
            pltpu.VMEM((K,), f32),
            pltpu.VMEM((ROWS_PT,), f32),
            pltpu.VMEM_SHARED((NPAD, 128), f32),
            pltpu.VMEM_SHARED((NPAD,), f32),
            pltpu.SemaphoreType.DMA,
            pltpu.SemaphoreType.DMA,
            pltpu.SemaphoreType.DMA,
            pltpu.SemaphoreType.DMA,
            pltpu.SemaphoreType.DMA,
            pltpu.SemaphoreType.DMA,
            pltpu.SemaphoreType.DMA,
        ],
    )(_sc_body)
    return fn(a0, a1, b0, b1, c_cat, src, dst)


def _tc3_body(h_ref, s0_ref, s1_ref, we2_ref, wu_ref, bu_ref, out_ref):
    agg = (jnp.dot(s0_ref[...], we2_ref[0:128, :],
                   preferred_element_type=jnp.float32)
           + jnp.dot(s1_ref[...], we2_ref[128:256, :],
                     preferred_element_type=jnp.float32))
    out_ref[...] = (jnp.dot(h_ref[...], wu_ref[0:DO, :],
                            preferred_element_type=jnp.float32)
                    + jnp.dot(agg, wu_ref[DO:2 * DO, :],
                              preferred_element_type=jnp.float32)
                    + bu_ref[...])


def _tc3(h_nodes, s0, s1, We2, Wu, bu):
    BN = 1000
    grid = (N // BN,)
    return pl.pallas_call(
        _tc3_body,
        grid=grid,
        in_specs=[
            pl.BlockSpec((BN, DO), lambda i: (i, 0)),
            pl.BlockSpec((BN, 128), lambda i: (i, 0)),
            pl.BlockSpec((BN, 128), lambda i: (i, 0)),
            pl.BlockSpec((DO, DO), lambda i: (0, 0)),
            pl.BlockSpec((2 * DO, DO), lambda i: (0, 0)),
            pl.BlockSpec((1, DO), lambda i: (0, 0)),
        ],
        out_specs=pl.BlockSpec((BN, DO), lambda i: (i, 0)),
        out_shape=jax.ShapeDtypeStruct((N, DO), jnp.float32),
    )(h_nodes, s0, s1, We2, Wu, bu.reshape(1, DO))


def kernel(x, edge_index, edge_attr, pos, Wn1, bn1, Wn2, bn2,
           We1, be1, We2, be2, Wu, bu):
    h_nodes, a0, a1, b0, b1 = _tc1(x, Wn1, bn1, Wn2, bn2, We1)
    src = edge_index[0]
    dst = edge_index[1]
    c_cat = _tc2(edge_attr, We1, be1)
    s_acc0, s_acc1, deg = _sc_edge(a0, a1, b0, b1, c_cat, src, dst)
    h_out = _tc3(h_nodes, s_acc0, s_acc1, We2, Wu, bu)
    h_out = h_out + deg[:N, None] * jnp.dot(be2, Wu[DO:2 * DO, :])
    return h_out
